# baseline (speedup 1.0000x reference)
import functools

import jax
import jax.numpy as jnp
from jax import lax
from jax.experimental import pallas as pl
from jax.experimental.pallas import tpu as pltpu
from jax.experimental.pallas import tpu_sc as plsc

B, T, V, D = 4096, 200, 1000, 64
VA = V + T
LANES = 16

_info = plsc.get_sparse_core_info()
NC, NS = _info.num_cores, _info.num_subcores
NW = NC * NS
SENT_PER_W = B // NW
CHUNK = T * D
NFULL = T // LANES
TAIL = T - NFULL * LANES
TPAD = (NFULL + 1) * LANES
POST_PAD = D * T + LANES


@functools.partial(
    pl.kernel,
    out_type=jax.ShapeDtypeStruct((B * T * D,), jnp.float32),
    mesh=plsc.VectorSubcoreMesh(core_axis_name="c", subcore_axis_name="s"),
    compiler_params=pltpu.CompilerParams(needs_layout_passes=False),
    scratch_types=[
        pltpu.VMEM((VA * D,), jnp.float32),
        pltpu.VMEM((POST_PAD,), jnp.float32),
        pltpu.VMEM((TPAD,), jnp.int32),
        pltpu.VMEM((TPAD,), jnp.int32),
        pltpu.VMEM((TPAD,), jnp.int32),
        pltpu.VMEM((TPAD,), jnp.int32),
        pltpu.VMEM((TPAD * D,), jnp.float32),
        pltpu.VMEM((TPAD * D,), jnp.float32),
        pltpu.SemaphoreType.DMA,
        pltpu.SemaphoreType.DMA,
        pltpu.SemaphoreType.DMA,
        pltpu.SemaphoreType.DMA,
        pltpu.SemaphoreType.DMA,
        pltpu.SemaphoreType.DMA,
    ],
)
def _emb_kernel(aug_hbm, x_hbm, posT_hbm, out_hbm,
                tok_v, posT_v, x0, x1, x2, x3, ob0, ob1,
                sx0, sx1, sx2, sx3, so0, so1):
    wid = lax.axis_index("s") * NC + lax.axis_index("c")
    base = wid * SENT_PER_W

    xs = (x0, x1, x2, x3)
    sxs = (sx0, sx1, sx2, sx3)
    obs = (ob0, ob1)
    sos = (so0, so1)

    pltpu.sync_copy(aug_hbm, tok_v)
    pltpu.sync_copy(posT_hbm, posT_v)

    lane_iota = lax.iota(jnp.int32, LANES)

    def fire_x(c, slot):
        pltpu.async_copy(
            x_hbm.at[pl.ds((base + c) * T, T)], xs[slot].at[pl.ds(0, T)],
            sxs[slot])

    def wait_x(slot):
        pltpu.make_async_copy(
            x_hbm.at[pl.ds(0, T)], xs[slot].at[pl.ds(0, T)],
            sxs[slot]).wait()

    def fire_out(c, slot):
        pltpu.async_copy(
            obs[slot].at[pl.ds(0, CHUNK)],
            out_hbm.at[pl.ds((base + c) * CHUNK, CHUNK)], sos[slot])

    def wait_out(slot):
        pltpu.make_async_copy(
            obs[slot].at[pl.ds(0, CHUNK)],
            out_hbm.at[pl.ds(0, CHUNK)], sos[slot]).wait()

    def compute(xslot, oslot):
        xv, slab = xs[xslot], obs[oslot]

        def group(g, carry):
            xg = xv[pl.ds(g * LANES, LANES)]
            tv = g * LANES + lane_iota
            idxg = jnp.where(xg == 2, V + tv, xg)
            bvec = idxg * D
            rowoff = tv * D
            for c in range(D):
                tok = plsc.load_gather(tok_v, [bvec + c])
                pos = posT_v[pl.ds(c * T + g * LANES, LANES)]
                plsc.store_scatter(slab, [rowoff + c], tok + pos)
            return carry

        lax.fori_loop(0, NFULL, group, 0)

        g = NFULL
        xg = xv[pl.ds(g * LANES, LANES)]
        tv = g * LANES + lane_iota
        idxg = jnp.where(xg == 2, V + tv, xg)
        bvec = jnp.clip(idxg, 0, VA - 1) * D
        rowoff = tv * D
        for c in range(D):
            tok = plsc.load_gather(tok_v, [bvec + c])
            pos = posT_v[pl.ds(c * T + g * LANES, LANES)]
            plsc.store_scatter(slab, [rowoff + c], tok + pos)

    fire_x(0, 0)
    fire_x(1, 1)
    fire_x(2, 2)
    fire_x(3, 3)

    def super_step(g, carry):
        for k in (0, 1, 2, 3):
            c = g * 4 + k
            oslot = k % 2

            @pl.when(c >= 2)
            def _():
                wait_out(oslot)

            wait_x(k)
            compute(k, oslot)

            @pl.when(c + 4 < SENT_PER_W)
            def _():
                fire_x(c + 4, k)

            fire_out(c, oslot)
        return carry

    lax.fori_loop(0, SENT_PER_W // 4, super_step, 0)

    wait_out(0)
    wait_out(1)


def kernel(x, start_token, end_token, tok_table, pos_table):
    aug = jnp.concatenate([tok_table, jnp.float32(-5.0) - pos_table], axis=0)
    posT = jnp.pad(pos_table.T.reshape(-1), (0, LANES))
    out = _emb_kernel(
        aug.reshape(-1), x.reshape(-1).astype(jnp.int32), posT
    )
    return out.reshape(B, T, D)


# speedup vs baseline: 3.5939x; 3.5939x over previous
import functools

import jax
import jax.numpy as jnp
from jax import lax
from jax.experimental import pallas as pl
from jax.experimental.pallas import tpu as pltpu
from jax.experimental.pallas import tpu_sc as plsc

B, T, V, D = 4096, 200, 1000, 64
VA = V + T
LANES = 16
JJ = D // LANES

_info = plsc.get_sparse_core_info()
NC, NS = _info.num_cores, _info.num_subcores
NW = NC * NS
SENT_PER_W = B // NW
WORDS_PER_W = SENT_PER_W * T
NGRP = WORDS_PER_W // LANES
G0 = 128
G1 = T - G0


@functools.partial(
    pl.kernel,
    out_type=jax.ShapeDtypeStruct((B * T, D), jnp.float32),
    mesh=plsc.VectorSubcoreMesh(core_axis_name="c", subcore_axis_name="s"),
    compiler_params=pltpu.CompilerParams(use_tc_tiling_on_sc=False),
    scratch_types=[
        pltpu.VMEM((T * D,), jnp.float32),
        pltpu.VMEM((WORDS_PER_W,), jnp.int32),
        pltpu.VMEM((T, D), jnp.float32),
        pltpu.VMEM((T, D), jnp.float32),
        pltpu.VMEM((T, D), jnp.float32),
        pltpu.VMEM((T, D), jnp.float32),
        pltpu.SemaphoreType.DMA,
        pltpu.SemaphoreType.DMA,
        pltpu.SemaphoreType.DMA,
        pltpu.SemaphoreType.DMA,
        pltpu.SemaphoreType.DMA,
        pltpu.SemaphoreType.DMA,
        pltpu.SemaphoreType.DMA,
        pltpu.SemaphoreType.DMA,
        pltpu.SemaphoreType.DMA,
    ],
)
def _emb_kernel(aug_hbm, x_hbm, pos_hbm, out_hbm,
                pos_v, idx_v, r0, r1, r2, r3,
                si, sg0, sg1, sg2, sg3, so0, so1, so2, so3):
    wid = lax.axis_index("s") * NC + lax.axis_index("c")
    base = wid * SENT_PER_W

    rows = (r0, r1, r2, r3)
    sgs = (sg0, sg1, sg2, sg3)
    sos = (so0, so1, so2, so3)

    pltpu.async_copy(x_hbm.at[pl.ds(base * T, WORDS_PER_W)], idx_v, si)
    pltpu.sync_copy(pos_hbm, pos_v)
    pltpu.make_async_copy(
        x_hbm.at[pl.ds(0, WORDS_PER_W)], idx_v, si).wait()

    lane_iota = lax.iota(jnp.int32, LANES)

    def grp(k, carry):
        xg = idx_v[pl.ds(k * LANES, LANES)]
        tv = lax.rem(k * LANES + lane_iota, T)
        idxg = jnp.where(xg == 2, V + tv, xg)
        idx_v[pl.ds(k * LANES, LANES)] = jnp.clip(idxg, 0, VA - 1) + wid * VA
        return carry

    lax.fori_loop(0, NGRP, grp, 0, unroll=4)

    def fire_gather(c, slot):
        pltpu.async_copy(
            aug_hbm.at[idx_v.at[pl.ds(c * T, T)]], rows[slot], sgs[slot])

    def wait_gather(slot):
        pltpu.make_async_copy(
            aug_hbm.at[idx_v.at[pl.ds(0, T)]], rows[slot], sgs[slot]).wait()

    def add_pos(slot):
        rv = rows[slot]

        def row(r, carry):
            rbase = r * D
            for jj in range(JJ):
                sl = pl.ds(jj * LANES, LANES)
                plsc.addupdate(
                    rv.at[r, sl], pos_v[pl.ds(rbase + jj * LANES, LANES)])
            return carry

        lax.fori_loop(0, T, row, 0, unroll=4)

    def fire_out(c, slot):
        pltpu.async_copy(
            rows[slot], out_hbm.at[pl.ds((base + c) * T, T)], sos[slot])

    def wait_out(slot):
        pltpu.make_async_copy(
            rows[slot], out_hbm.at[pl.ds(0, T)], sos[slot]).wait()

    fire_gather(0, 0)
    fire_gather(1, 1)
    fire_gather(2, 2)

    def super_step(g, carry):
        for slot in (0, 1, 2, 3):
            c = g * 4 + slot
            nslot = (slot + 3) % 4

            @pl.when(c >= 1)
            def _():
                wait_out(nslot)

            @pl.when(c + 3 < SENT_PER_W)
            def _():
                fire_gather(c + 3, nslot)

            wait_gather(slot)
            add_pos(slot)
            fire_out(c, slot)
        return carry

    lax.fori_loop(0, SENT_PER_W // 4, super_step, 0)

    wait_out(3)


def kernel(x, start_token, end_token, tok_table, pos_table):
    aug = jnp.concatenate([tok_table, jnp.float32(-5.0) - pos_table], axis=0)
    aug = jnp.tile(aug, (NW, 1))
    out = _emb_kernel(
        aug, x.reshape(-1).astype(jnp.int32), pos_table.reshape(-1)
    )
    return out.reshape(B, T, D)


# speedup vs baseline: 3.7956x; 1.0561x over previous
import functools

import jax
import jax.numpy as jnp
from jax import lax
from jax.experimental import pallas as pl
from jax.experimental.pallas import tpu as pltpu
from jax.experimental.pallas import tpu_sc as plsc

B, T, V, D = 4096, 200, 1000, 64
VA = V + T
LANES = 16
JJ = D // LANES

_info = plsc.get_sparse_core_info()
NC, NS = _info.num_cores, _info.num_subcores
NW = NC * NS
SENT_PER_W = B // NW
WORDS_PER_W = SENT_PER_W * T
NGRP = WORDS_PER_W // LANES
G0 = 128
G1 = T - G0


@functools.partial(
    pl.kernel,
    out_type=jax.ShapeDtypeStruct((B * T, D), jnp.float32),
    mesh=plsc.VectorSubcoreMesh(core_axis_name="c", subcore_axis_name="s"),
    compiler_params=pltpu.CompilerParams(use_tc_tiling_on_sc=False),
    scratch_types=[
        pltpu.VMEM((T * D,), jnp.float32),
        pltpu.VMEM((WORDS_PER_W,), jnp.int32),
        pltpu.VMEM((T, D), jnp.float32),
        pltpu.VMEM((T, D), jnp.float32),
        pltpu.VMEM((T, D), jnp.float32),
        pltpu.VMEM((T, D), jnp.float32),
        pltpu.SemaphoreType.DMA,
        pltpu.SemaphoreType.DMA,
        pltpu.SemaphoreType.DMA,
        pltpu.SemaphoreType.DMA,
        pltpu.SemaphoreType.DMA,
        pltpu.SemaphoreType.DMA,
        pltpu.SemaphoreType.DMA,
        pltpu.SemaphoreType.DMA,
        pltpu.SemaphoreType.DMA,
        pltpu.SemaphoreType.DMA,
        pltpu.SemaphoreType.DMA,
        pltpu.SemaphoreType.DMA,
        pltpu.SemaphoreType.DMA,
    ],
)
def _emb_kernel(aug_hbm, x_hbm, pos_hbm, out_hbm,
                pos_v, idx_v, r0, r1, r2, r3,
                si, sg0, sg1, sg2, sg3, sh0, sh1, sh2, sh3,
                so0, so1, so2, so3):
    wid = lax.axis_index("s") * NC + lax.axis_index("c")
    base = wid * SENT_PER_W

    rows = (r0, r1, r2, r3)
    sgs = (sg0, sg1, sg2, sg3)
    shs = (sh0, sh1, sh2, sh3)
    sos = (so0, so1, so2, so3)

    pltpu.async_copy(x_hbm.at[pl.ds(base * T, WORDS_PER_W)], idx_v, si)
    pltpu.sync_copy(pos_hbm, pos_v)
    pltpu.make_async_copy(
        x_hbm.at[pl.ds(0, WORDS_PER_W)], idx_v, si).wait()

    lane_iota = lax.iota(jnp.int32, LANES)

    def grp(k, carry):
        xg = idx_v[pl.ds(k * LANES, LANES)]
        tv = lax.rem(k * LANES + lane_iota, T)
        idxg = jnp.where(xg == 2, V + tv, xg)
        idx_v[pl.ds(k * LANES, LANES)] = jnp.clip(idxg, 0, VA - 1) + wid * VA
        return carry

    lax.fori_loop(0, NGRP, grp, 0, unroll=4)

    HA, HB = 104, 96

    def fire_gather(c, slot):
        rv = rows[slot]
        pltpu.async_copy(
            aug_hbm.at[idx_v.at[pl.ds(c * T, HA)]],
            rv.at[pl.ds(0, HA)], sgs[slot])
        pltpu.async_copy(
            aug_hbm.at[idx_v.at[pl.ds(c * T + HA, HB)]],
            rv.at[pl.ds(HA, HB)], shs[slot])

    def wait_gather(slot):
        rv = rows[slot]
        pltpu.make_async_copy(
            aug_hbm.at[idx_v.at[pl.ds(0, HA)]],
            rv.at[pl.ds(0, HA)], sgs[slot]).wait()
        pltpu.make_async_copy(
            aug_hbm.at[idx_v.at[pl.ds(0, HB)]],
            rv.at[pl.ds(HA, HB)], shs[slot]).wait()

    def add_pos(slot):
        rv = rows[slot]

        def row(r, carry):
            rbase = r * D
            for jj in range(JJ):
                sl = pl.ds(jj * LANES, LANES)
                plsc.addupdate(
                    rv.at[r, sl], pos_v[pl.ds(rbase + jj * LANES, LANES)])
            return carry

        lax.fori_loop(0, T, row, 0, unroll=4)

    def fire_out(c, slot):
        pltpu.async_copy(
            rows[slot], out_hbm.at[pl.ds((base + c) * T, T)], sos[slot])

    def wait_out(slot):
        pltpu.make_async_copy(
            rows[slot], out_hbm.at[pl.ds(0, T)], sos[slot]).wait()

    fire_gather(0, 0)
    fire_gather(1, 1)

    def super_step(g, carry):
        for slot in (0, 1, 2, 3):
            c = g * 4 + slot
            nslot = (slot + 2) % 4

            @pl.when(c >= 2)
            def _():
                wait_out(nslot)

            @pl.when(c + 2 < SENT_PER_W)
            def _():
                fire_gather(c + 2, nslot)

            wait_gather(slot)
            add_pos(slot)
            fire_out(c, slot)
        return carry

    lax.fori_loop(0, SENT_PER_W // 4, super_step, 0)

    wait_out(2)
    wait_out(3)


def kernel(x, start_token, end_token, tok_table, pos_table):
    aug = jnp.concatenate([tok_table, jnp.float32(-5.0) - pos_table], axis=0)
    aug = jnp.tile(aug, (NW, 1))
    out = _emb_kernel(
        aug, x.reshape(-1).astype(jnp.int32), pos_table.reshape(-1)
    )
    return out.reshape(B, T, D)
